# Initial kernel scaffold; baseline (speedup 1.0000x reference)
import jax
import jax.numpy as jnp
from jax.experimental import pallas as pl


def kernel(op_idx, a, b, c, op_embed, Wp, bp, keys, tpos, W1, b1, W2, b2, Wh1, bh1, Wh2, bh2):
    raise NotImplementedError("write your pallas kernel here")


# speedup vs baseline: 2.2696x; 2.2696x over previous
import functools

import jax
import jax.numpy as jnp
from jax.experimental import pallas as pl
from jax.experimental.pallas import tpu as pltpu

B = 4096
D = 256
T = 16
L = 2
F = 2 * D
TOPK = 4
SPREAD = 1.5

BT = 512
NB = B // BT


def _moe_kernel(opi_ref, a_ref, b_ref, c_ref, featw_ref, bp_ref, keyst_ref,
                tpos_ref, w1_ref, b1_ref, w2_ref, b2_ref, wh1_ref, bh1_ref,
                wh2_ref, bh2_ref, res_ref, topi_ref, part_ref):
    f32 = jnp.float32
    opi = opi_ref[...]
    iota8 = jax.lax.broadcasted_iota(jnp.int32, (BT, 8), 1)
    onehot = (opi == iota8).astype(f32)
    a_bits = ((a_ref[...] >> iota8) & 1).astype(f32)
    b_bits = ((b_ref[...] >> iota8) & 1).astype(f32)
    c_f = c_ref[...].astype(f32)

    x = (jnp.dot(onehot, featw_ref[0], preferred_element_type=f32)
         + jnp.dot(a_bits, featw_ref[1], preferred_element_type=f32)
         + jnp.dot(b_bits, featw_ref[2], preferred_element_type=f32)
         + c_f * featw_ref[3, 0:1, :]
         + bp_ref[...])

    pos = opi.astype(f32)
    iota16 = jax.lax.broadcasted_iota(jnp.int32, (BT, T), 1)
    inv_sqrt_d = 1.0 / (D ** 0.5)
    topi_cols = None
    for l in range(L):
        content = jnp.dot(x, keyst_ref[l], preferred_element_type=f32) * inv_sqrt_d
        delta = pos - tpos_ref[l]
        logits = content - delta * delta * (1.0 / (2.0 * SPREAD * SPREAD))
        m = jnp.max(logits, axis=1, keepdims=True)
        e = jnp.exp(logits - m)
        s = jnp.sum(e, axis=1, keepdims=True)
        probs = e / s

        p = probs
        topi_cols = []
        topv_cols = []
        for _ in range(TOPK):
            mx = jnp.max(p, axis=1, keepdims=True)
            idx = jnp.min(jnp.where(p == mx, iota16, T), axis=1, keepdims=True)
            topi_cols.append(idx)
            topv_cols.append(mx)
            p = jnp.where(iota16 == idx, -jnp.inf, p)
        gsum = topv_cols[0] + topv_cols[1] + topv_cols[2] + topv_cols[3]
        inv_gsum = 1.0 / (gsum + 1e-9)
        gate_full = jnp.zeros((BT, T), f32)
        for k in range(TOPK):
            gate_full = gate_full + jnp.where(
                iota16 == topi_cols[k], topv_cols[k] * inv_gsum, 0.0)

        xb = x.astype(jnp.bfloat16)
        acc = jnp.dot(gate_full, b2_ref[l], preferred_element_type=f32)
        for t in range(T):
            h = jnp.dot(xb, w1_ref[l, t], preferred_element_type=f32)
            h = jnp.maximum(h + b1_ref[l, t], 0.0)
            hb = (h * gate_full[:, t:t + 1]).astype(jnp.bfloat16)
            acc = acc + jnp.dot(hb, w2_ref[l, t], preferred_element_type=f32)
        x = x + acc

        part_ref[0, l, 0:1, :] = jnp.sum(probs, axis=0, keepdims=True)
        part_ref[0, l, 1:2, :] = jnp.sum((gate_full > 0).astype(f32), axis=0,
                                         keepdims=True)

    hh = jnp.maximum(jnp.dot(x, wh1_ref[...], preferred_element_type=f32)
                     + bh1_ref[...], 0.0)
    res_ref[...] = jax.nn.sigmoid(
        jnp.dot(hh, wh2_ref[...], preferred_element_type=f32) + bh2_ref[...])
    for k in range(TOPK):
        topi_ref[:, k:k + 1] = topi_cols[k]


@functools.partial(jax.jit, static_argnames=())
def kernel(op_idx, a, b, c, op_embed, Wp, bp, keys, tpos, W1, b1, W2, b2,
           Wh1, bh1, Wh2, bh2):
    f32 = jnp.float32
    opi2 = op_idx.astype(jnp.int32).reshape(B, 1)
    a2 = a.astype(jnp.int32).reshape(B, 1)
    b2_ = b.astype(jnp.int32).reshape(B, 1)
    c2 = c.astype(jnp.int32).reshape(B, 1)

    featw = jnp.stack([
        op_embed @ Wp[0:32],
        Wp[32:40],
        Wp[40:48],
        jnp.concatenate([Wp[48:49]] * 8, axis=0),
    ], axis=0)
    bp2 = bp.reshape(1, D)
    keys_t = keys.transpose(0, 2, 1)
    tpos3 = tpos.reshape(L, 1, T)
    w1b = W1.astype(jnp.bfloat16)
    w2b = W2.astype(jnp.bfloat16)
    b13 = b1.reshape(L, T, 1, F)
    wh1 = Wh1
    bh1_2 = bh1.reshape(1, 64)
    wh2 = Wh2
    bh2_2 = bh2.reshape(1, 8)

    const = lambda shape: pl.BlockSpec(shape, lambda i: (0,) * len(shape))
    res, topi, part = pl.pallas_call(
        _moe_kernel,
        grid=(NB,),
        in_specs=[
            pl.BlockSpec((BT, 1), lambda i: (i, 0)),
            pl.BlockSpec((BT, 1), lambda i: (i, 0)),
            pl.BlockSpec((BT, 1), lambda i: (i, 0)),
            pl.BlockSpec((BT, 1), lambda i: (i, 0)),
            const((4, 8, D)),
            const((1, D)),
            const((L, D, T)),
            const((L, 1, T)),
            const((L, T, D, F)),
            const((L, T, 1, F)),
            const((L, T, F, D)),
            const((L, T, D)),
            const((D, 64)),
            const((1, 64)),
            const((64, 8)),
            const((1, 8)),
        ],
        out_specs=[
            pl.BlockSpec((BT, 8), lambda i: (i, 0)),
            pl.BlockSpec((BT, TOPK), lambda i: (i, 0)),
            pl.BlockSpec((1, L, 2, T), lambda i: (i, 0, 0, 0)),
        ],
        out_shape=[
            jax.ShapeDtypeStruct((B, 8), f32),
            jax.ShapeDtypeStruct((B, TOPK), jnp.int32),
            jax.ShapeDtypeStruct((NB, L, 2, T), f32),
        ],
        compiler_params=pltpu.CompilerParams(
            dimension_semantics=("arbitrary",)),
    )(opi2, a2, b2_, c2, featw, bp2, keys_t, tpos3, w1b, b13, w2b, b2,
      wh1, bh1_2, wh2, bh2_2)

    sums = jnp.sum(part, axis=0)
    importance = sums[:, 0, :] / B
    load = sums[:, 1, :] / B
    total_aux = jnp.sum(T * jnp.sum(importance * load, axis=-1))
    return res, topi, total_aux.astype(f32)
